# baseline (speedup 1.0000x reference)
import jax
import jax.numpy as jnp
from jax.experimental import pallas as pl
from jax.experimental.pallas import tpu as pltpu

_Q = 65536
_B = 1024
_D = 128
_CH = 2048
_NCH = _Q // _CH
_K = 8
_A = 4


def _body(new_ref, keys_ref, old_ref, upd_ref, hbuf, hsem, *rest):
    bufs, gsems, ssems = rest[:_K], rest[_K:2 * _K], rest[2 * _K:]

    pltpu.make_async_copy(new_ref, hbuf, hsem).start()

    def gather(ci):
        return pltpu.make_async_copy(
            keys_ref.at[pl.ds(ci * _CH, _CH)], bufs[ci % _K], gsems[ci % _K])

    def scatters(ci):
        ws = [pltpu.make_async_copy(
            bufs[ci % _K], old_ref.at[pl.ds(ci * _CH, _CH)], ssems[ci % _K])]
        lo = ci * _CH + _B
        rows = min(_CH, _Q - lo)
        if rows > 0:
            ws.append(pltpu.make_async_copy(
                bufs[ci % _K].at[pl.ds(0, rows)],
                upd_ref.at[pl.ds(lo, rows)], ssems[ci % _K]))
        return ws

    gathers = {}
    pending = {}
    for ci in range(min(_A, _NCH)):
        gathers[ci] = gather(ci)
        gathers[ci].start()

    hdone = False
    for ci in range(_NCH):
        nf = ci + _A
        if nf < _NCH:
            if nf - _K >= 0:
                for w in pending.pop(nf - _K):
                    w.wait()
            gathers[nf] = gather(nf)
            gathers[nf].start()
        gathers.pop(ci).wait()
        ws = scatters(ci)
        for w in ws:
            w.start()
        pending[ci] = ws
        if not hdone:
            pltpu.make_async_copy(new_ref, hbuf, hsem).wait()
            pltpu.make_async_copy(hbuf, upd_ref.at[pl.ds(0, _B)], hsem).start()
            hdone = True

    pltpu.make_async_copy(hbuf, upd_ref.at[pl.ds(0, _B)], hsem).wait()
    for ci in sorted(pending):
        for w in pending[ci]:
            w.wait()


def kernel(new_keys, keys):
    old, upd = pl.pallas_call(
        _body,
        in_specs=[
            pl.BlockSpec(memory_space=pl.ANY),
            pl.BlockSpec(memory_space=pl.ANY),
        ],
        out_specs=[
            pl.BlockSpec(memory_space=pl.ANY),
            pl.BlockSpec(memory_space=pl.ANY),
        ],
        out_shape=[
            jax.ShapeDtypeStruct((_Q, _D), jnp.float32),
            jax.ShapeDtypeStruct((_Q, _D), jnp.float32),
        ],
        scratch_shapes=(
            [pltpu.VMEM((_B, _D), jnp.float32), pltpu.SemaphoreType.DMA]
            + [pltpu.VMEM((_CH, _D), jnp.float32) for _ in range(_K)]
            + [pltpu.SemaphoreType.DMA for _ in range(2 * _K)]
        ),
    )(new_keys, keys)
    return (old, upd)


# speedup vs baseline: 1.0749x; 1.0749x over previous
import jax
import jax.numpy as jnp
from jax.experimental import pallas as pl
from jax.experimental.pallas import tpu as pltpu

_Q = 65536
_B = 1024
_D = 128
_CH = 16384
_NCH = _Q // _CH
_K = 3
_A = 2


def _body(new_ref, keys_ref, old_ref, upd_ref, hbuf, hsem, *rest):
    bufs, gsems, ssems = rest[:_K], rest[_K:2 * _K], rest[2 * _K:]

    pltpu.make_async_copy(new_ref, hbuf, hsem).start()

    def gather(ci):
        return pltpu.make_async_copy(
            keys_ref.at[pl.ds(ci * _CH, _CH)], bufs[ci % _K], gsems[ci % _K])

    def scatters(ci):
        ws = [pltpu.make_async_copy(
            bufs[ci % _K], old_ref.at[pl.ds(ci * _CH, _CH)], ssems[ci % _K])]
        lo = ci * _CH + _B
        rows = min(_CH, _Q - lo)
        if rows > 0:
            ws.append(pltpu.make_async_copy(
                bufs[ci % _K].at[pl.ds(0, rows)],
                upd_ref.at[pl.ds(lo, rows)], ssems[ci % _K]))
        return ws

    gathers = {}
    pending = {}
    for ci in range(min(_A, _NCH)):
        gathers[ci] = gather(ci)
        gathers[ci].start()

    hdone = False
    for ci in range(_NCH):
        nf = ci + _A
        if nf < _NCH:
            if nf - _K >= 0:
                for w in pending.pop(nf - _K):
                    w.wait()
            gathers[nf] = gather(nf)
            gathers[nf].start()
        gathers.pop(ci).wait()
        ws = scatters(ci)
        for w in ws:
            w.start()
        pending[ci] = ws
        if not hdone:
            pltpu.make_async_copy(new_ref, hbuf, hsem).wait()
            pltpu.make_async_copy(hbuf, upd_ref.at[pl.ds(0, _B)], hsem).start()
            hdone = True

    pltpu.make_async_copy(hbuf, upd_ref.at[pl.ds(0, _B)], hsem).wait()
    for ci in sorted(pending):
        for w in pending[ci]:
            w.wait()


def kernel(new_keys, keys):
    old, upd = pl.pallas_call(
        _body,
        in_specs=[
            pl.BlockSpec(memory_space=pl.ANY),
            pl.BlockSpec(memory_space=pl.ANY),
        ],
        out_specs=[
            pl.BlockSpec(memory_space=pl.ANY),
            pl.BlockSpec(memory_space=pl.ANY),
        ],
        out_shape=[
            jax.ShapeDtypeStruct((_Q, _D), jnp.float32),
            jax.ShapeDtypeStruct((_Q, _D), jnp.float32),
        ],
        scratch_shapes=(
            [pltpu.VMEM((_B, _D), jnp.float32), pltpu.SemaphoreType.DMA]
            + [pltpu.VMEM((_CH, _D), jnp.float32) for _ in range(_K)]
            + [pltpu.SemaphoreType.DMA for _ in range(2 * _K)]
        ),
    )(new_keys, keys)
    return (old, upd)


# speedup vs baseline: 1.1000x; 1.0233x over previous
import jax
import jax.numpy as jnp
from jax.experimental import pallas as pl
from jax.experimental.pallas import tpu as pltpu

_Q = 65536
_B = 1024
_D = 128
_CH = 32768
_NCH = _Q // _CH
_K = 2
_A = 1


def _body(new_ref, keys_ref, old_ref, upd_ref, hbuf, hsem, *rest):
    bufs, gsems, ssems = rest[:_K], rest[_K:2 * _K], rest[2 * _K:]

    pltpu.make_async_copy(new_ref, hbuf, hsem).start()

    def gather(ci):
        return pltpu.make_async_copy(
            keys_ref.at[pl.ds(ci * _CH, _CH)], bufs[ci % _K], gsems[ci % _K])

    def scatters(ci):
        ws = [pltpu.make_async_copy(
            bufs[ci % _K], old_ref.at[pl.ds(ci * _CH, _CH)], ssems[ci % _K])]
        lo = ci * _CH + _B
        rows = min(_CH, _Q - lo)
        if rows > 0:
            ws.append(pltpu.make_async_copy(
                bufs[ci % _K].at[pl.ds(0, rows)],
                upd_ref.at[pl.ds(lo, rows)], ssems[ci % _K]))
        return ws

    gathers = {}
    pending = {}
    for ci in range(min(_A, _NCH)):
        gathers[ci] = gather(ci)
        gathers[ci].start()

    hdone = False
    for ci in range(_NCH):
        nf = ci + _A
        if nf < _NCH:
            if nf - _K >= 0:
                for w in pending.pop(nf - _K):
                    w.wait()
            gathers[nf] = gather(nf)
            gathers[nf].start()
        gathers.pop(ci).wait()
        ws = scatters(ci)
        for w in ws:
            w.start()
        pending[ci] = ws
        if not hdone:
            pltpu.make_async_copy(new_ref, hbuf, hsem).wait()
            pltpu.make_async_copy(hbuf, upd_ref.at[pl.ds(0, _B)], hsem).start()
            hdone = True

    pltpu.make_async_copy(hbuf, upd_ref.at[pl.ds(0, _B)], hsem).wait()
    for ci in sorted(pending):
        for w in pending[ci]:
            w.wait()


def kernel(new_keys, keys):
    old, upd = pl.pallas_call(
        _body,
        in_specs=[
            pl.BlockSpec(memory_space=pl.ANY),
            pl.BlockSpec(memory_space=pl.ANY),
        ],
        out_specs=[
            pl.BlockSpec(memory_space=pl.ANY),
            pl.BlockSpec(memory_space=pl.ANY),
        ],
        out_shape=[
            jax.ShapeDtypeStruct((_Q, _D), jnp.float32),
            jax.ShapeDtypeStruct((_Q, _D), jnp.float32),
        ],
        scratch_shapes=(
            [pltpu.VMEM((_B, _D), jnp.float32), pltpu.SemaphoreType.DMA]
            + [pltpu.VMEM((_CH, _D), jnp.float32) for _ in range(_K)]
            + [pltpu.SemaphoreType.DMA for _ in range(2 * _K)]
        ),
    )(new_keys, keys)
    return (old, upd)
